# baseline (speedup 1.0000x reference)
import functools

import jax
import jax.numpy as jnp
from jax import lax
from jax.experimental import pallas as pl
from jax.experimental.pallas import tpu as pltpu
from jax.experimental.pallas import tpu_sc as plsc

_N = 100000
_E = 1600000
_G = 64
_L = 128
_NTILE = 16
_NP = 100352
_STRIPE = _NP // _NTILE
_EGP = 12672
_EPAD = _EGP * _L
_K = 6

_RB = 2048
_NBLK = _NP // _RB

_mesh = plsc.VectorSubcoreMesh(core_axis_name="c", subcore_axis_name="s")


def _prelu(x, a):
    return jnp.maximum(x, 0.0) + a * jnp.minimum(x, 0.0)


def _ln(x, w, b, eps=1e-5):
    mu = jnp.mean(x, axis=-1, keepdims=True)
    var = jnp.mean((x - mu) * (x - mu), axis=-1, keepdims=True)
    return (x - mu) * lax.rsqrt(var + eps) * w + b


def _deg_body(ep_hbm, ones_hbm, z16_hbm, out_hbm, idxa_v, idxb_v, ones_v, acc_sh, sem, semb):
    c = lax.axis_index("c")
    s = lax.axis_index("s")
    row = s * _STRIPE
    pltpu.sync_copy(z16_hbm.at[pl.ds(row, _STRIPE)], acc_sh.at[pl.ds(row, _STRIPE)])
    pltpu.sync_copy(ones_hbm, ones_v)
    plsc.subcore_barrier()
    npg = _EGP // 2 // _NTILE
    g0 = c * (_EGP // 2) + s * npg

    @pl.loop(0, npg, step=2 * _K)
    def _(i):
        lda = pltpu.async_copy(ep_hbm.at[1, pl.ds(g0 + i, _K)], idxa_v, sem)
        ldb = pltpu.async_copy(ep_hbm.at[1, pl.ds(g0 + i + _K, _K)], idxb_v, semb)
        lda.wait()
        sa = [
            pltpu.async_copy(ones_v, acc_sh.at[idxa_v.at[j]], sem, add=True)
            for j in range(_K)
        ]
        ldb.wait()
        for d in sa:
            d.wait()
        sb = [
            pltpu.async_copy(ones_v, acc_sh.at[idxb_v.at[j]], semb, add=True)
            for j in range(_K)
        ]
        for d in sb:
            d.wait()

    plsc.subcore_barrier()
    pltpu.sync_copy(acc_sh.at[pl.ds(row, _STRIPE)], out_hbm.at[c, pl.ds(row, _STRIPE)])


def _edge_loop(ep_hbm, tab, acc_sh, bufs, g0, npg):
    isa, ida, isb, idb, msga, msgb, sema, semb, semg, semh = bufs

    @pl.loop(0, npg, step=2 * _K)
    def _(i):
        ga = g0 + i
        gb = g0 + i + _K
        lsa = pltpu.async_copy(ep_hbm.at[0, pl.ds(ga, _K)], isa, sema)
        lda = pltpu.async_copy(ep_hbm.at[1, pl.ds(ga, _K)], ida, sema)
        lsb = pltpu.async_copy(ep_hbm.at[0, pl.ds(gb, _K)], isb, semb)
        ldb = pltpu.async_copy(ep_hbm.at[1, pl.ds(gb, _K)], idb, semb)
        lsa.wait()
        gas = [
            pltpu.async_copy(tab.at[isa.at[j]], msga.at[pl.ds(j * _L, _L)], semg)
            for j in range(_K)
        ]
        lda.wait()
        for d in gas:
            d.wait()
        sas = [
            pltpu.async_copy(msga.at[pl.ds(j * _L, _L)], acc_sh.at[ida.at[j]], sema, add=True)
            for j in range(_K)
        ]
        lsb.wait()
        gbs = [
            pltpu.async_copy(tab.at[isb.at[j]], msgb.at[pl.ds(j * _L, _L)], semh)
            for j in range(_K)
        ]
        ldb.wait()
        for d in gbs:
            d.wait()
        for d in sas:
            d.wait()
        sbs = [
            pltpu.async_copy(msgb.at[pl.ds(j * _L, _L)], acc_sh.at[idb.at[j]], semb, add=True)
            for j in range(_K)
        ]
        for d in sbs:
            d.wait()


def _agg1_body(ep_hbm, xs_hbm, z16_hbm, out_hbm, isa, ida, isb, idb, msga, msgb,
               acc_sh, sema, semb, semg, semh):
    c = lax.axis_index("c")
    s = lax.axis_index("s")
    row = s * _STRIPE
    pltpu.sync_copy(z16_hbm.at[pl.ds(row, _STRIPE)], acc_sh.at[pl.ds(row, _STRIPE)])
    plsc.subcore_barrier()
    npg = _EGP // 2 // _NTILE
    g0 = c * (_EGP // 2) + s * npg
    _edge_loop(ep_hbm, xs_hbm, acc_sh,
               (isa, ida, isb, idb, msga, msgb, sema, semb, semg, semh), g0, npg)
    plsc.subcore_barrier()
    pltpu.sync_copy(acc_sh.at[pl.ds(row, _STRIPE)], out_hbm.at[c, pl.ds(row, _STRIPE)])


def _agg16_body(ep_hbm, y2_hbm, z16_hbm, out_hbm, isa, ida, isb, idb, msga, msgb,
                acc_sh, sema, semb, semg, semh):
    c = lax.axis_index("c")
    s = lax.axis_index("s")
    row = s * _STRIPE
    pltpu.sync_copy(z16_hbm.at[pl.ds(row, _STRIPE)], acc_sh.at[pl.ds(row, _STRIPE)])
    plsc.subcore_barrier()
    npg = _EGP // _NTILE
    g0 = s * npg
    _edge_loop(ep_hbm, y2_hbm.at[c], acc_sh,
               (isa, ida, isb, idb, msga, msgb, sema, semb, semg, semh), g0, npg)
    plsc.subcore_barrier()
    pltpu.sync_copy(acc_sh.at[pl.ds(row, _STRIPE)], out_hbm.at[c, pl.ds(row, _STRIPE)])


_sc_params = pltpu.CompilerParams(use_tc_tiling_on_sc=False)

_sc_deg = functools.partial(
    pl.kernel,
    out_type=jax.ShapeDtypeStruct((2, _NP, 16), jnp.float32),
    mesh=_mesh,
    compiler_params=_sc_params,
    scratch_types=[
        pltpu.VMEM((_K, _L), jnp.int32),
        pltpu.VMEM((_K, _L), jnp.int32),
        pltpu.VMEM((_L, 16), jnp.float32),
        pltpu.VMEM_SHARED((_NP, 16), jnp.float32),
        pltpu.SemaphoreType.DMA,
        pltpu.SemaphoreType.DMA,
    ],
)(_deg_body)

_sc_agg1 = functools.partial(
    pl.kernel,
    out_type=jax.ShapeDtypeStruct((2, _NP, 16), jnp.float32),
    mesh=_mesh,
    compiler_params=_sc_params,
    scratch_types=[
        pltpu.VMEM((_K, _L), jnp.int32),
        pltpu.VMEM((_K, _L), jnp.int32),
        pltpu.VMEM((_K, _L), jnp.int32),
        pltpu.VMEM((_K, _L), jnp.int32),
        pltpu.VMEM((_K * _L, 16), jnp.float32),
        pltpu.VMEM((_K * _L, 16), jnp.float32),
        pltpu.VMEM_SHARED((_NP, 16), jnp.float32),
        pltpu.SemaphoreType.DMA,
        pltpu.SemaphoreType.DMA,
        pltpu.SemaphoreType.DMA,
        pltpu.SemaphoreType.DMA,
    ],
)(_agg1_body)

_sc_agg16 = functools.partial(
    pl.kernel,
    out_type=jax.ShapeDtypeStruct((2, _NP, 16), jnp.float32),
    mesh=_mesh,
    compiler_params=_sc_params,
    scratch_types=[
        pltpu.VMEM((_K, _L), jnp.int32),
        pltpu.VMEM((_K, _L), jnp.int32),
        pltpu.VMEM((_K, _L), jnp.int32),
        pltpu.VMEM((_K, _L), jnp.int32),
        pltpu.VMEM((_K * _L, 16), jnp.float32),
        pltpu.VMEM((_K * _L, 16), jnp.float32),
        pltpu.VMEM_SHARED((_NP, 16), jnp.float32),
        pltpu.SemaphoreType.DMA,
        pltpu.SemaphoreType.DMA,
        pltpu.SemaphoreType.DMA,
        pltpu.SemaphoreType.DMA,
    ],
)(_agg16_body)


def _dinv16(dp_ref):
    return lax.rsqrt(dp_ref[0] + dp_ref[1] + 1.0)


def _prep_body(x4_ref, dp_ref, xs_ref):
    dinv = _dinv16(dp_ref)
    xs_ref[...] = jnp.concatenate(
        [x4_ref[...] * dinv[:, :4], jnp.zeros((_RB, 12), jnp.float32)], axis=1)


def _tc_prep(x4, degp):
    return pl.pallas_call(
        _prep_body,
        grid=(_NBLK,),
        in_specs=[
            pl.BlockSpec((_RB, 4), lambda i: (i, 0)),
            pl.BlockSpec((2, _RB, 16), lambda i: (0, i, 0)),
        ],
        out_specs=pl.BlockSpec((_RB, 16), lambda i: (i, 0)),
        out_shape=jax.ShapeDtypeStruct((_NP, 16), jnp.float32),
    )(x4, degp)


def _post1_body(acc_ref, xs_ref, dp_ref, w_ref, b_ref, lnw_ref, lnb_ref, a_ref, y2_ref):
    dinv = _dinv16(dp_ref)
    s4 = (acc_ref[0] + acc_ref[1] + xs_ref[...]) * dinv
    h = jnp.dot(s4, w_ref[...], preferred_element_type=jnp.float32) + b_ref[...]
    y = _prelu(_ln(h, lnw_ref[...], lnb_ref[...]), a_ref[0, 0])
    y2_ref[0] = y[:, :16] * dinv
    y2_ref[1] = y[:, 16:] * dinv


def _tc_post1(acc1, xs4, degp, w1p, b1, lnw, lnb, a1):
    return pl.pallas_call(
        _post1_body,
        grid=(_NBLK,),
        in_specs=[
            pl.BlockSpec((2, _RB, 16), lambda i: (0, i, 0)),
            pl.BlockSpec((_RB, 16), lambda i: (i, 0)),
            pl.BlockSpec((2, _RB, 16), lambda i: (0, i, 0)),
            pl.BlockSpec((16, 32), lambda i: (0, 0)),
            pl.BlockSpec((1, 32), lambda i: (0, 0)),
            pl.BlockSpec((1, 32), lambda i: (0, 0)),
            pl.BlockSpec((1, 32), lambda i: (0, 0)),
            pl.BlockSpec((1, 1), lambda i: (0, 0)),
        ],
        out_specs=pl.BlockSpec((2, _RB, 16), lambda i: (0, i, 0)),
        out_shape=jax.ShapeDtypeStruct((2, _NP, 16), jnp.float32),
    )(acc1, xs4, degp, w1p, b1, lnw, lnb, a1)


def _post2_body(acc_ref, y2_ref, dp_ref, bat_ref, w2_ref, b2_ref, lnw_ref, lnb_ref,
                a2_ref, mw1_ref, mb1_ref, mlnw_ref, mlnb_ref, ma_ref, mw2_ref, mb2_ref,
                out_ref, sums_ref, cnt_ref):
    i = pl.program_id(0)

    @pl.when(i == 0)
    def _():
        sums_ref[...] = jnp.zeros_like(sums_ref)
        cnt_ref[...] = jnp.zeros_like(cnt_ref)

    dinv = _dinv16(dp_ref)
    h32 = jnp.concatenate([(acc_ref[0] + y2_ref[0]) * dinv,
                           (acc_ref[1] + y2_ref[1]) * dinv], axis=1)
    h = jnp.dot(h32, w2_ref[...], preferred_element_type=jnp.float32) + b2_ref[...]
    y = _prelu(_ln(h, lnw_ref[...], lnb_ref[...]), a2_ref[0, 0])
    ids = lax.broadcasted_iota(jnp.int32, (_G, _RB), 0)
    mt = (bat_ref[0] == ids).astype(jnp.float32)
    sums_ref[...] += lax.dot_general(mt, y, (((1,), (0,)), ((), ())),
                                     preferred_element_type=jnp.float32)
    cnt_ref[...] += lax.dot_general(mt, jnp.ones((_RB, 1), jnp.float32),
                                    (((1,), (0,)), ((), ())),
                                    preferred_element_type=jnp.float32)

    @pl.when(i == _NBLK - 1)
    def _():
        pooled = sums_ref[...] / jnp.maximum(cnt_ref[...], 1.0)
        p = jnp.dot(pooled, mw1_ref[...], preferred_element_type=jnp.float32) + mb1_ref[...]
        p = _prelu(_ln(p, mlnw_ref[...], mlnb_ref[...]), ma_ref[0, 0])
        out_ref[...] = jnp.dot(p, mw2_ref[...], preferred_element_type=jnp.float32) + mb2_ref[...]


def _tc_post2(acc2, y2, degp, batr, w2, b2, lnw, lnb, a2, mw1, mb1, mlnw, mlnb, ma, mw2, mb2):
    full = lambda r, c: pl.BlockSpec((r, c), lambda i: (0, 0))
    return pl.pallas_call(
        _post2_body,
        grid=(_NBLK,),
        in_specs=[
            pl.BlockSpec((2, _RB, 16), lambda i: (0, i, 0)),
            pl.BlockSpec((2, _RB, 16), lambda i: (0, i, 0)),
            pl.BlockSpec((2, _RB, 16), lambda i: (0, i, 0)),
            pl.BlockSpec((1, 1, _RB), lambda i: (i, 0, 0)),
            full(32, 32), full(1, 32), full(1, 32), full(1, 32), full(1, 1),
            full(32, 16), full(1, 16), full(1, 16), full(1, 16), full(1, 1),
            full(16, 10), full(1, 10),
        ],
        out_specs=pl.BlockSpec((_G, 10), lambda i: (0, 0)),
        out_shape=jax.ShapeDtypeStruct((_G, 10), jnp.float32),
        scratch_shapes=[
            pltpu.VMEM((_G, 32), jnp.float32),
            pltpu.VMEM((_G, 1), jnp.float32),
        ],
    )(acc2, y2, degp, batr, w2, b2, lnw, lnb, a2, mw1, mb1, mlnw, mlnb, ma, mw2, mb2)


def kernel(x, edge_index, batch, W1, b1, ln1_w, ln1_b, a1, W2, b2, ln2_w, ln2_b, a2,
           mW1, mb1, mln_w, mln_b, ma, mW2, mb2):
    f32 = jnp.float32
    ei = jnp.pad(edge_index, ((0, 0), (0, _EPAD - _E)), constant_values=_NP - 1)
    ep = ei.reshape(2, _EGP, _L)
    x4 = jnp.pad(x, ((0, _NP - _N), (0, 1)))
    batr = jnp.pad(batch, (0, _NP - _N),
                   constant_values=_G).reshape(_NBLK, 1, _RB)
    z16 = jnp.zeros((_NP, 16), f32)
    ones16 = jnp.ones((_L, 16), f32)
    w1p = jnp.pad(W1, ((0, 13), (0, 0)))

    degp = _sc_deg(ep, ones16, z16)
    xs16 = _tc_prep(x4, degp)
    acc1 = _sc_agg1(ep, xs16, z16)
    y2 = _tc_post1(acc1, xs16, degp, w1p, b1.reshape(1, 32),
                   ln1_w.reshape(1, 32), ln1_b.reshape(1, 32), a1.reshape(1, 1))
    acc2 = _sc_agg16(ep, y2, z16)
    return _tc_post2(acc2, y2, degp, batr, W2, b2.reshape(1, 32),
                     ln2_w.reshape(1, 32), ln2_b.reshape(1, 32), a2.reshape(1, 1),
                     mW1, mb1.reshape(1, 16), mln_w.reshape(1, 16),
                     mln_b.reshape(1, 16), ma.reshape(1, 1), mW2, mb2.reshape(1, 10))


# speedup vs baseline: 1.0828x; 1.0828x over previous
import functools

import jax
import jax.numpy as jnp
from jax import lax
from jax.experimental import pallas as pl
from jax.experimental.pallas import tpu as pltpu
from jax.experimental.pallas import tpu_sc as plsc

_N = 100000
_E = 1600000
_G = 64
_L = 128
_NTILE = 16
_NP = 100352
_STRIPE = _NP // _NTILE
_EGP = 12544
_EPAD = _EGP * _L
_K = 4

_RB = 2048
_NBLK = _NP // _RB

_mesh = plsc.VectorSubcoreMesh(core_axis_name="c", subcore_axis_name="s")


def _prelu(x, a):
    return jnp.maximum(x, 0.0) + a * jnp.minimum(x, 0.0)


def _ln(x, w, b, eps=1e-5):
    mu = jnp.mean(x, axis=-1, keepdims=True)
    var = jnp.mean((x - mu) * (x - mu), axis=-1, keepdims=True)
    return (x - mu) * lax.rsqrt(var + eps) * w + b


def _deg_body(ep_hbm, ones_hbm, z16_hbm, out_hbm, idxa_v, idxb_v, ones_v, acc_sh, sem, semb):
    c = lax.axis_index("c")
    s = lax.axis_index("s")
    row = s * _STRIPE
    pltpu.sync_copy(z16_hbm.at[pl.ds(row, _STRIPE)], acc_sh.at[pl.ds(row, _STRIPE)])
    pltpu.sync_copy(ones_hbm, ones_v)
    plsc.subcore_barrier()
    npg = _EGP // 2 // _NTILE
    g0 = c * (_EGP // 2) + s * npg

    @pl.loop(0, npg, step=2 * _K)
    def _(i):
        lda = pltpu.async_copy(ep_hbm.at[1, pl.ds(g0 + i, _K)], idxa_v, sem)
        ldb = pltpu.async_copy(ep_hbm.at[1, pl.ds(g0 + i + _K, _K)], idxb_v, semb)
        lda.wait()
        sa = [
            pltpu.async_copy(ones_v, acc_sh.at[idxa_v.at[j]], sem, add=True)
            for j in range(_K)
        ]
        ldb.wait()
        for d in sa:
            d.wait()
        sb = [
            pltpu.async_copy(ones_v, acc_sh.at[idxb_v.at[j]], semb, add=True)
            for j in range(_K)
        ]
        for d in sb:
            d.wait()

    plsc.subcore_barrier()
    pltpu.sync_copy(acc_sh.at[pl.ds(row, _STRIPE)], out_hbm.at[c, pl.ds(row, _STRIPE)])


def _edge_loop(ep_hbm, tab, acc_sh, bufs, g0, npg):
    isa, ida, isb, idb, msga, msgb, sema, semb, semg, semh = bufs

    @pl.loop(0, npg, step=2 * _K)
    def _(i):
        ga = g0 + i
        gb = g0 + i + _K
        lsa = pltpu.async_copy(ep_hbm.at[0, pl.ds(ga, _K)], isa, sema)
        lda = pltpu.async_copy(ep_hbm.at[1, pl.ds(ga, _K)], ida, sema)
        lsb = pltpu.async_copy(ep_hbm.at[0, pl.ds(gb, _K)], isb, semb)
        ldb = pltpu.async_copy(ep_hbm.at[1, pl.ds(gb, _K)], idb, semb)
        lsa.wait()
        gas = [
            pltpu.async_copy(tab.at[isa.at[j]], msga.at[pl.ds(j * _L, _L)], semg)
            for j in range(_K)
        ]
        lda.wait()
        for d in gas:
            d.wait()
        sas = [
            pltpu.async_copy(msga.at[pl.ds(j * _L, _L)], acc_sh.at[ida.at[j]], sema, add=True)
            for j in range(_K)
        ]
        lsb.wait()
        gbs = [
            pltpu.async_copy(tab.at[isb.at[j]], msgb.at[pl.ds(j * _L, _L)], semh)
            for j in range(_K)
        ]
        ldb.wait()
        for d in gbs:
            d.wait()
        for d in sas:
            d.wait()
        sbs = [
            pltpu.async_copy(msgb.at[pl.ds(j * _L, _L)], acc_sh.at[idb.at[j]], semb, add=True)
            for j in range(_K)
        ]
        for d in sbs:
            d.wait()


def _agg1_body(ep_hbm, xs_hbm, z16_hbm, out_hbm, isa, ida, isb, idb, msga, msgb,
               acc_sh, sema, semb, semg, semh):
    c = lax.axis_index("c")
    s = lax.axis_index("s")
    row = s * _STRIPE
    pltpu.sync_copy(z16_hbm.at[pl.ds(row, _STRIPE)], acc_sh.at[pl.ds(row, _STRIPE)])
    plsc.subcore_barrier()
    npg = _EGP // 2 // _NTILE
    g0 = c * (_EGP // 2) + s * npg
    _edge_loop(ep_hbm, xs_hbm, acc_sh,
               (isa, ida, isb, idb, msga, msgb, sema, semb, semg, semh), g0, npg)
    plsc.subcore_barrier()
    pltpu.sync_copy(acc_sh.at[pl.ds(row, _STRIPE)], out_hbm.at[c, pl.ds(row, _STRIPE)])


def _agg16_body(ep_hbm, y2_hbm, z16_hbm, out_hbm, isa, ida, isb, idb, msga, msgb,
                acc_sh, sema, semb, semg, semh):
    c = lax.axis_index("c")
    s = lax.axis_index("s")
    row = s * _STRIPE
    pltpu.sync_copy(z16_hbm.at[pl.ds(row, _STRIPE)], acc_sh.at[pl.ds(row, _STRIPE)])
    plsc.subcore_barrier()
    npg = _EGP // _NTILE
    g0 = s * npg
    _edge_loop(ep_hbm, y2_hbm.at[c], acc_sh,
               (isa, ida, isb, idb, msga, msgb, sema, semb, semg, semh), g0, npg)
    plsc.subcore_barrier()
    pltpu.sync_copy(acc_sh.at[pl.ds(row, _STRIPE)], out_hbm.at[c, pl.ds(row, _STRIPE)])


_sc_params = pltpu.CompilerParams(use_tc_tiling_on_sc=False)

_sc_deg = functools.partial(
    pl.kernel,
    out_type=jax.ShapeDtypeStruct((2, _NP, 16), jnp.float32),
    mesh=_mesh,
    compiler_params=_sc_params,
    scratch_types=[
        pltpu.VMEM((_K, _L), jnp.int32),
        pltpu.VMEM((_K, _L), jnp.int32),
        pltpu.VMEM((_L, 16), jnp.float32),
        pltpu.VMEM_SHARED((_NP, 16), jnp.float32),
        pltpu.SemaphoreType.DMA,
        pltpu.SemaphoreType.DMA,
    ],
)(_deg_body)

_sc_agg1 = functools.partial(
    pl.kernel,
    out_type=jax.ShapeDtypeStruct((2, _NP, 16), jnp.float32),
    mesh=_mesh,
    compiler_params=_sc_params,
    scratch_types=[
        pltpu.VMEM((_K, _L), jnp.int32),
        pltpu.VMEM((_K, _L), jnp.int32),
        pltpu.VMEM((_K, _L), jnp.int32),
        pltpu.VMEM((_K, _L), jnp.int32),
        pltpu.VMEM((_K * _L, 16), jnp.float32),
        pltpu.VMEM((_K * _L, 16), jnp.float32),
        pltpu.VMEM_SHARED((_NP, 16), jnp.float32),
        pltpu.SemaphoreType.DMA,
        pltpu.SemaphoreType.DMA,
        pltpu.SemaphoreType.DMA,
        pltpu.SemaphoreType.DMA,
    ],
)(_agg1_body)

_sc_agg16 = functools.partial(
    pl.kernel,
    out_type=jax.ShapeDtypeStruct((2, _NP, 16), jnp.float32),
    mesh=_mesh,
    compiler_params=_sc_params,
    scratch_types=[
        pltpu.VMEM((_K, _L), jnp.int32),
        pltpu.VMEM((_K, _L), jnp.int32),
        pltpu.VMEM((_K, _L), jnp.int32),
        pltpu.VMEM((_K, _L), jnp.int32),
        pltpu.VMEM((_K * _L, 16), jnp.float32),
        pltpu.VMEM((_K * _L, 16), jnp.float32),
        pltpu.VMEM_SHARED((_NP, 16), jnp.float32),
        pltpu.SemaphoreType.DMA,
        pltpu.SemaphoreType.DMA,
        pltpu.SemaphoreType.DMA,
        pltpu.SemaphoreType.DMA,
    ],
)(_agg16_body)


def _dinv16(dp_ref):
    return lax.rsqrt(dp_ref[0] + dp_ref[1] + 1.0)


def _prep_body(x4_ref, dp_ref, xs_ref):
    dinv = _dinv16(dp_ref)
    xs_ref[...] = jnp.concatenate(
        [x4_ref[...] * dinv[:, :4], jnp.zeros((_RB, 12), jnp.float32)], axis=1)


def _tc_prep(x4, degp):
    return pl.pallas_call(
        _prep_body,
        grid=(_NBLK,),
        in_specs=[
            pl.BlockSpec((_RB, 4), lambda i: (i, 0)),
            pl.BlockSpec((2, _RB, 16), lambda i: (0, i, 0)),
        ],
        out_specs=pl.BlockSpec((_RB, 16), lambda i: (i, 0)),
        out_shape=jax.ShapeDtypeStruct((_NP, 16), jnp.float32),
    )(x4, degp)


def _post1_body(acc_ref, xs_ref, dp_ref, w_ref, b_ref, lnw_ref, lnb_ref, a_ref, y2_ref):
    dinv = _dinv16(dp_ref)
    s4 = (acc_ref[0] + acc_ref[1] + xs_ref[...]) * dinv
    h = jnp.dot(s4, w_ref[...], preferred_element_type=jnp.float32) + b_ref[...]
    y = _prelu(_ln(h, lnw_ref[...], lnb_ref[...]), a_ref[0, 0])
    y2_ref[0] = y[:, :16] * dinv
    y2_ref[1] = y[:, 16:] * dinv


def _tc_post1(acc1, xs4, degp, w1p, b1, lnw, lnb, a1):
    return pl.pallas_call(
        _post1_body,
        grid=(_NBLK,),
        in_specs=[
            pl.BlockSpec((2, _RB, 16), lambda i: (0, i, 0)),
            pl.BlockSpec((_RB, 16), lambda i: (i, 0)),
            pl.BlockSpec((2, _RB, 16), lambda i: (0, i, 0)),
            pl.BlockSpec((16, 32), lambda i: (0, 0)),
            pl.BlockSpec((1, 32), lambda i: (0, 0)),
            pl.BlockSpec((1, 32), lambda i: (0, 0)),
            pl.BlockSpec((1, 32), lambda i: (0, 0)),
            pl.BlockSpec((1, 1), lambda i: (0, 0)),
        ],
        out_specs=pl.BlockSpec((2, _RB, 16), lambda i: (0, i, 0)),
        out_shape=jax.ShapeDtypeStruct((2, _NP, 16), jnp.float32),
    )(acc1, xs4, degp, w1p, b1, lnw, lnb, a1)


def _post2_body(acc_ref, y2_ref, dp_ref, bat_ref, w2_ref, b2_ref, lnw_ref, lnb_ref,
                a2_ref, mw1_ref, mb1_ref, mlnw_ref, mlnb_ref, ma_ref, mw2_ref, mb2_ref,
                out_ref, sums_ref, cnt_ref):
    i = pl.program_id(0)

    @pl.when(i == 0)
    def _():
        sums_ref[...] = jnp.zeros_like(sums_ref)
        cnt_ref[...] = jnp.zeros_like(cnt_ref)

    dinv = _dinv16(dp_ref)
    h32 = jnp.concatenate([(acc_ref[0] + y2_ref[0]) * dinv,
                           (acc_ref[1] + y2_ref[1]) * dinv], axis=1)
    h = jnp.dot(h32, w2_ref[...], preferred_element_type=jnp.float32) + b2_ref[...]
    y = _prelu(_ln(h, lnw_ref[...], lnb_ref[...]), a2_ref[0, 0])
    ids = lax.broadcasted_iota(jnp.int32, (_G, _RB), 0)
    mt = (bat_ref[0] == ids).astype(jnp.float32)
    sums_ref[...] += lax.dot_general(mt, y, (((1,), (0,)), ((), ())),
                                     preferred_element_type=jnp.float32)
    cnt_ref[...] += lax.dot_general(mt, jnp.ones((_RB, 1), jnp.float32),
                                    (((1,), (0,)), ((), ())),
                                    preferred_element_type=jnp.float32)

    @pl.when(i == _NBLK - 1)
    def _():
        pooled = sums_ref[...] / jnp.maximum(cnt_ref[...], 1.0)
        p = jnp.dot(pooled, mw1_ref[...], preferred_element_type=jnp.float32) + mb1_ref[...]
        p = _prelu(_ln(p, mlnw_ref[...], mlnb_ref[...]), ma_ref[0, 0])
        out_ref[...] = jnp.dot(p, mw2_ref[...], preferred_element_type=jnp.float32) + mb2_ref[...]


def _tc_post2(acc2, y2, degp, batr, w2, b2, lnw, lnb, a2, mw1, mb1, mlnw, mlnb, ma, mw2, mb2):
    full = lambda r, c: pl.BlockSpec((r, c), lambda i: (0, 0))
    return pl.pallas_call(
        _post2_body,
        grid=(_NBLK,),
        in_specs=[
            pl.BlockSpec((2, _RB, 16), lambda i: (0, i, 0)),
            pl.BlockSpec((2, _RB, 16), lambda i: (0, i, 0)),
            pl.BlockSpec((2, _RB, 16), lambda i: (0, i, 0)),
            pl.BlockSpec((1, 1, _RB), lambda i: (i, 0, 0)),
            full(32, 32), full(1, 32), full(1, 32), full(1, 32), full(1, 1),
            full(32, 16), full(1, 16), full(1, 16), full(1, 16), full(1, 1),
            full(16, 10), full(1, 10),
        ],
        out_specs=pl.BlockSpec((_G, 10), lambda i: (0, 0)),
        out_shape=jax.ShapeDtypeStruct((_G, 10), jnp.float32),
        scratch_shapes=[
            pltpu.VMEM((_G, 32), jnp.float32),
            pltpu.VMEM((_G, 1), jnp.float32),
        ],
    )(acc2, y2, degp, batr, w2, b2, lnw, lnb, a2, mw1, mb1, mlnw, mlnb, ma, mw2, mb2)


def kernel(x, edge_index, batch, W1, b1, ln1_w, ln1_b, a1, W2, b2, ln2_w, ln2_b, a2,
           mW1, mb1, mln_w, mln_b, ma, mW2, mb2):
    f32 = jnp.float32
    ei = jnp.pad(edge_index, ((0, 0), (0, _EPAD - _E)), constant_values=_NP - 1)
    ep = ei.reshape(2, _EGP, _L)
    x4 = jnp.pad(x, ((0, _NP - _N), (0, 1)))
    batr = jnp.pad(batch, (0, _NP - _N),
                   constant_values=_G).reshape(_NBLK, 1, _RB)
    z16 = jnp.zeros((_NP, 16), f32)
    ones16 = jnp.ones((_L, 16), f32)
    w1p = jnp.pad(W1, ((0, 13), (0, 0)))

    degp = _sc_deg(ep, ones16, z16)
    xs16 = _tc_prep(x4, degp)
    acc1 = _sc_agg1(ep, xs16, z16)
    y2 = _tc_post1(acc1, xs16, degp, w1p, b1.reshape(1, 32),
                   ln1_w.reshape(1, 32), ln1_b.reshape(1, 32), a1.reshape(1, 1))
    acc2 = _sc_agg16(ep, y2, z16)
    return _tc_post2(acc2, y2, degp, batr, W2, b2.reshape(1, 32),
                     ln2_w.reshape(1, 32), ln2_b.reshape(1, 32), a2.reshape(1, 1),
                     mW1, mb1.reshape(1, 16), mln_w.reshape(1, 16),
                     mln_b.reshape(1, 16), ma.reshape(1, 1), mW2, mb2.reshape(1, 10))
